# Initial kernel scaffold; baseline (speedup 1.0000x reference)
import jax
import jax.numpy as jnp
from jax import lax
from jax.experimental import pallas as pl
from jax.experimental.pallas import tpu as pltpu
from jax.experimental.pallas import tpu_sc as plsc

BS, AN, CN, OD = 4096, 200, 18, 32
NC, NS, L = 2, 16, 16
NW = NC * NS
B_PER_W = BS // NW
NB = 8
NCHUNK = B_PER_W // NB
NG = (AN + L - 1) // L


def _body(x_hbm, emb_hbm, out_hbm, x_v, emb_v, out_v):
    wid = lax.axis_index("c") * NS + lax.axis_index("s")
    b_base = wid * B_PER_W

    pltpu.sync_copy(emb_hbm, emb_v)
    iota = lax.iota(jnp.int32, L)

    def chunk_body(ci, _):
        b0 = b_base + ci * NB
        pltpu.sync_copy(x_hbm.at[pl.ds(b0, NB)], x_v)

        def b_body(b, _):
            bvec = jnp.broadcast_to(b, (L,))

            def g_body(g, _):
                a0 = jnp.minimum(g * L, AN - L)
                avec = a0 + iota
                m = plsc.load_gather(x_v, [bvec, avec, jnp.zeros((L,), jnp.int32)])
                best = jnp.zeros((L,), jnp.int32)
                for c in range(1, CN):
                    v = plsc.load_gather(
                        x_v, [bvec, avec, jnp.full((L,), c, jnp.int32)])
                    gt = v > m
                    m = jnp.where(gt, v, m)
                    best = jnp.where(gt, jnp.full((L,), c, jnp.int32), best)
                for d in range(OD):
                    dvec = jnp.full((L,), d, jnp.int32)
                    val = plsc.load_gather(emb_v, [best, dvec])
                    plsc.store_scatter(out_v, [avec, bvec, dvec], val)
                return ()

            lax.fori_loop(0, NG, g_body, ())
            return ()

        lax.fori_loop(0, NB, b_body, ())
        pltpu.sync_copy(out_v, out_hbm.at[:, pl.ds(b0, NB), :])
        return ()

    lax.fori_loop(0, NCHUNK, chunk_body, ())


@jax.jit
def kernel(x, emb):
    mesh = plsc.VectorSubcoreMesh(core_axis_name="c", subcore_axis_name="s")
    f = pl.kernel(
        _body,
        out_type=jax.ShapeDtypeStruct((AN, BS, OD), jnp.float32),
        mesh=mesh,
        scratch_types=[
            pltpu.VMEM((NB, AN, CN), jnp.float32),
            pltpu.VMEM((CN, OD), jnp.float32),
            pltpu.VMEM((AN, NB, OD), jnp.float32),
        ],
    )
    return f(x, emb)


# speedup vs baseline: 1.2720x; 1.2720x over previous
import jax
import jax.numpy as jnp
from jax import lax
from jax.experimental import pallas as pl
from jax.experimental.pallas import tpu as pltpu
from jax.experimental.pallas import tpu_sc as plsc

BS, AN, CN, OD = 4096, 200, 18, 32
NC, NS, L = 2, 16, 16
NW = NC * NS
B_PER_W = BS // NW
NB = 8
NCHUNK = B_PER_W // NB
NG = (AN + L - 1) // L


def _body(x_hbm, emb_hbm, out_hbm, x_v, emb_v, out_v):
    wid = lax.axis_index("c") * NS + lax.axis_index("s")
    b_base = wid * B_PER_W

    pltpu.sync_copy(emb_hbm, emb_v)
    iota = lax.iota(jnp.int32, L)

    def chunk_body(ci, _):
        b0 = b_base + ci * NB
        pltpu.sync_copy(x_hbm.at[pl.ds(b0, NB)], x_v)

        def b_body(b, _):
            bvec = jnp.broadcast_to(b, (L,))

            def g_body(g, _):
                a0 = jnp.minimum(g * L, AN - L)
                avec = a0 + iota
                m = plsc.load_gather(x_v, [bvec, avec, jnp.zeros((L,), jnp.int32)])
                best = jnp.zeros((L,), jnp.int32)
                for c in range(1, CN):
                    v = plsc.load_gather(
                        x_v, [bvec, avec, jnp.full((L,), c, jnp.int32)])
                    gt = v > m
                    m = jnp.where(gt, v, m)
                    best = jnp.where(gt, jnp.full((L,), c, jnp.int32), best)
                for d in range(OD):
                    dvec = jnp.full((L,), d, jnp.int32)
                    val = plsc.load_gather(emb_v, [best, dvec])
                    plsc.store_scatter(out_v, [avec, bvec, dvec], val)
                return ()

            lax.fori_loop(0, NG, g_body, ())
            return ()

        lax.fori_loop(0, NB, b_body, ())
        pltpu.sync_copy(out_v, out_hbm.at[:, pl.ds(b0, NB), :])
        return ()

    lax.fori_loop(0, NCHUNK, chunk_body, ())


@jax.jit
def kernel(x, emb):
    mesh = plsc.VectorSubcoreMesh(core_axis_name="c", subcore_axis_name="s")
    f = pl.kernel(
        _body,
        out_type=jax.ShapeDtypeStruct((AN, BS, OD), jnp.float32),
        mesh=mesh,
        scratch_types=[
            pltpu.VMEM((NB, AN, CN), jnp.float32),
            pltpu.VMEM((CN, OD), jnp.float32),
            pltpu.VMEM((AN, NB, OD), jnp.float32),
        ],
        compiler_params=pltpu.CompilerParams(
            use_tc_tiling_on_sc=False, needs_layout_passes=False),
    )
    return f(x, emb)


# speedup vs baseline: 3.9258x; 3.0864x over previous
import jax
import jax.numpy as jnp
from jax import lax
from jax.experimental import pallas as pl
from jax.experimental.pallas import tpu as pltpu
from jax.experimental.pallas import tpu_sc as plsc

BS, AN, CN, OD = 4096, 200, 18, 32
NC, NS, L = 2, 16, 16
NW = NC * NS
BT = BS // NW
NA = 8
NCHUNK = AN // NA
NGRP = BT // L


def _body(x_hbm, emb_hbm, out_hbm, x_v, emb_v, out_v):
    wid = lax.axis_index("c") * NS + lax.axis_index("s")
    b0 = wid * BT

    pltpu.sync_copy(emb_hbm, emb_v)
    iota = lax.iota(jnp.int32, L)

    def chunk_body(ci, _):
        a0 = ci * NA
        pltpu.sync_copy(x_hbm.at[:, pl.ds(a0, NA), pl.ds(b0, BT)], x_v)

        def a_body(a, _):
            avec = jnp.broadcast_to(a, (L,))

            def g_body(g, _):
                bvec = g * L + iota
                m = plsc.load_gather(
                    x_v, [jnp.zeros((L,), jnp.int32), avec, bvec])
                best = jnp.zeros((L,), jnp.int32)
                for c in range(1, CN):
                    v = plsc.load_gather(
                        x_v, [jnp.full((L,), c, jnp.int32), avec, bvec])
                    gt = v > m
                    m = jnp.where(gt, v, m)
                    best = jnp.where(gt, jnp.full((L,), c, jnp.int32), best)
                for d in range(OD):
                    val = plsc.load_gather(
                        emb_v, [best, jnp.full((L,), d, jnp.int32)])
                    out_v[a, d, pl.ds(g * L, L)] = val
                return ()

            lax.fori_loop(0, NGRP, g_body, ())
            return ()

        lax.fori_loop(0, NA, a_body, ())
        pltpu.sync_copy(out_v, out_hbm.at[pl.ds(a0, NA), :, pl.ds(b0, BT)])
        return ()

    lax.fori_loop(0, NCHUNK, chunk_body, ())


@jax.jit
def kernel(x, emb):
    mesh = plsc.VectorSubcoreMesh(core_axis_name="c", subcore_axis_name="s")
    f = pl.kernel(
        _body,
        out_type=jax.ShapeDtypeStruct((AN, OD, BS), jnp.float32),
        mesh=mesh,
        scratch_types=[
            pltpu.VMEM((CN, NA, BT), jnp.float32),
            pltpu.VMEM((CN, OD), jnp.float32),
            pltpu.VMEM((NA, OD, BT), jnp.float32),
        ],
        compiler_params=pltpu.CompilerParams(
            use_tc_tiling_on_sc=True, needs_layout_passes=False),
    )
    x_t = jnp.transpose(x, (2, 1, 0))
    out_t = f(x_t, emb)
    return jnp.transpose(out_t, (0, 2, 1))
